# Initial kernel scaffold; baseline (speedup 1.0000x reference)
import functools

import jax
import jax.numpy as jnp
from jax import lax
from jax.experimental import pallas as pl
from jax.experimental.pallas import tpu as pltpu
from jax.experimental.pallas import tpu_sc as plsc

N_NODES = 50000
HIDDEN = 64
NUM_CLASSES = 4
CLIN_DIM = 128
MEL_DIM = 512
N_EDGES = 800000

NC = 2
NS = 16
NPAD = 51200
ROWS_PER_SUB = NPAD // NS
HALF = HIDDEN // 2
L2W = 16

R = 1000
GRID = N_NODES // R


def _mesh():
    return plsc.VectorSubcoreMesh(core_axis_name="c", subcore_axis_name="s")


def _deg_body(dst_hbm, zero1_hbm, out_hbm, acc, idx_v, ones_v, *, epw, chunk):
    c = lax.axis_index("c")
    s = lax.axis_index("s")
    r0 = s * ROWS_PER_SUB
    pltpu.sync_copy(zero1_hbm, acc.at[pl.ds(r0, ROWS_PER_SUB)])
    def fill(i, _):
        ones_v[pl.ds(i * 16, 16)] = jnp.ones((16,), jnp.float32)
        return 0
    lax.fori_loop(0, chunk // 16, fill, 0)
    plsc.subcore_barrier()

    base = (c * NS + s) * epw

    def step(k, _):
        off = base + k * chunk
        pltpu.sync_copy(dst_hbm.at[pl.ds(off, chunk)], idx_v)
        pltpu.sync_copy(ones_v, acc.at[idx_v], add=True)
        return 0
    lax.fori_loop(0, epw // chunk, step, 0)

    plsc.subcore_barrier()
    pltpu.sync_copy(acc.at[pl.ds(r0, ROWS_PER_SUB)],
                    out_hbm.at[c, pl.ds(r0, ROWS_PER_SUB)])


def _degree(dst_i32):
    epw = N_EDGES // (NC * NS)
    chunk = 5000
    body = functools.partial(_deg_body, epw=epw, chunk=chunk)
    f = pl.kernel(
        body,
        out_type=jax.ShapeDtypeStruct((NC, NPAD), jnp.float32),
        mesh=_mesh(),
        scratch_types=[
            pltpu.VMEM_SHARED((NPAD,), jnp.float32),
            pltpu.VMEM((chunk,), jnp.int32),
            pltpu.VMEM((chunk,), jnp.float32),
        ],
        name="sc_degree",
    )
    zero1 = jnp.zeros((ROWS_PER_SUB,), jnp.float32)
    return f(dst_i32, zero1)


def _mp1_body(src_hbm, dst_hbm, hsa_hbm, hsb_hbm, zero2_hbm,
              outa_hbm, outb_hbm,
              acc, src_v, dst_v, rows_v, gsem, *, eps, chunk):
    c = lax.axis_index("c")
    s = lax.axis_index("s")
    r0 = s * ROWS_PER_SUB
    pltpu.sync_copy(zero2_hbm, acc.at[pl.ds(r0, ROWS_PER_SUB)])
    plsc.subcore_barrier()

    base = s * eps
    nsteps = eps // chunk

    def make_step(hs_hbm):
        def step(k, _):
            off = base + k * chunk
            pltpu.sync_copy(src_hbm.at[pl.ds(off, chunk)], src_v)
            pltpu.sync_copy(dst_hbm.at[pl.ds(off, chunk)], dst_v)
            pltpu.async_copy(hs_hbm.at[src_v], rows_v, gsem).wait()
            pltpu.sync_copy(rows_v, acc.at[dst_v], add=True)
            return 0
        return step

    @pl.when(c == 0)
    def _():
        lax.fori_loop(0, nsteps, make_step(hsa_hbm), 0)

    @pl.when(c == 1)
    def _():
        lax.fori_loop(0, nsteps, make_step(hsb_hbm), 0)

    plsc.subcore_barrier()

    @pl.when(c == 0)
    def _():
        pltpu.sync_copy(acc.at[pl.ds(r0, ROWS_PER_SUB)],
                        outa_hbm.at[pl.ds(r0, ROWS_PER_SUB)])

    @pl.when(c == 1)
    def _():
        pltpu.sync_copy(acc.at[pl.ds(r0, ROWS_PER_SUB)],
                        outb_hbm.at[pl.ds(r0, ROWS_PER_SUB)])


def _message_pass1(src_i32, dst_i32, hs_a, hs_b):
    eps = N_EDGES // NS
    chunk = 1000
    body = functools.partial(_mp1_body, eps=eps, chunk=chunk)
    f = pl.kernel(
        body,
        out_type=(jax.ShapeDtypeStruct((NPAD, HALF), jnp.float32),
                  jax.ShapeDtypeStruct((NPAD, HALF), jnp.float32)),
        mesh=_mesh(),
        scratch_types=[
            pltpu.VMEM_SHARED((NPAD, HALF), jnp.float32),
            pltpu.VMEM((chunk,), jnp.int32),
            pltpu.VMEM((chunk,), jnp.int32),
            pltpu.VMEM((chunk, HALF), jnp.float32),
            pltpu.SemaphoreType.DMA,
        ],
        name="sc_message_pass1",
    )
    zero2 = jnp.zeros((ROWS_PER_SUB, HALF), jnp.float32)
    return f(src_i32, dst_i32, hs_a, hs_b, zero2)


def _mp2_body(src_hbm, dst_hbm, hs2_hbm, zero3_hbm, out_hbm,
              acc, src_v, dst_v, rows_v, gsem, *, epw, chunk):
    c = lax.axis_index("c")
    s = lax.axis_index("s")
    r0 = s * ROWS_PER_SUB
    pltpu.sync_copy(zero3_hbm, acc.at[pl.ds(r0, ROWS_PER_SUB)])
    plsc.subcore_barrier()

    base = (c * NS + s) * epw

    def step(k, _):
        off = base + k * chunk
        pltpu.sync_copy(src_hbm.at[pl.ds(off, chunk)], src_v)
        pltpu.sync_copy(dst_hbm.at[pl.ds(off, chunk)], dst_v)
        pltpu.async_copy(hs2_hbm.at[src_v], rows_v, gsem).wait()
        pltpu.sync_copy(rows_v, acc.at[dst_v], add=True)
        return 0
    lax.fori_loop(0, epw // chunk, step, 0)

    plsc.subcore_barrier()
    pltpu.sync_copy(acc.at[pl.ds(r0, ROWS_PER_SUB)],
                    out_hbm.at[c, pl.ds(r0, ROWS_PER_SUB)])


def _message_pass2(src_i32, dst_i32, hs2p):
    epw = N_EDGES // (NC * NS)
    chunk = 1000
    body = functools.partial(_mp2_body, epw=epw, chunk=chunk)
    f = pl.kernel(
        body,
        out_type=jax.ShapeDtypeStruct((NC, NPAD, L2W), jnp.float32),
        mesh=_mesh(),
        scratch_types=[
            pltpu.VMEM_SHARED((NPAD, L2W), jnp.float32),
            pltpu.VMEM((chunk,), jnp.int32),
            pltpu.VMEM((chunk,), jnp.int32),
            pltpu.VMEM((chunk, L2W), jnp.float32),
            pltpu.SemaphoreType.DMA,
        ],
        name="sc_message_pass2",
    )
    zero3 = jnp.zeros((ROWS_PER_SUB, L2W), jnp.float32)
    return f(src_i32, dst_i32, hs2p, zero3)


def _tc1_body(mel_ref, clin_ref, degp_ref, wm_ref, bm_ref, wcc_ref, wcm_ref,
              bc_ref, w1_ref, hsa_ref, hsb_ref, dinv_ref):
    m = jnp.maximum(
        jnp.dot(mel_ref[...], wm_ref[...], preferred_element_type=jnp.float32)
        + bm_ref[...], 0.0)
    x = jnp.maximum(
        jnp.dot(clin_ref[...], wcc_ref[...], preferred_element_type=jnp.float32)
        + jnp.dot(m, wcm_ref[...], preferred_element_type=jnp.float32)
        + bc_ref[...], 0.0)
    h1 = jnp.dot(x, w1_ref[...], preferred_element_type=jnp.float32)
    deg = degp_ref[0] + degp_ref[1] + 1.0
    dinv = lax.rsqrt(deg)
    hs = h1 * dinv
    hsa_ref[...] = hs[:, :HALF]
    hsb_ref[...] = hs[:, HALF:]
    dinv_ref[...] = dinv


def _tc1(mel, clinical, degp3, Wm, bm2, Wcc, Wcm, bc2, W1):
    return pl.pallas_call(
        _tc1_body,
        grid=(GRID,),
        in_specs=[
            pl.BlockSpec((R, MEL_DIM), lambda i: (i, 0)),
            pl.BlockSpec((R, CLIN_DIM), lambda i: (i, 0)),
            pl.BlockSpec((NC, R, 1), lambda i: (0, i, 0)),
            pl.BlockSpec((MEL_DIM, HIDDEN), lambda i: (0, 0)),
            pl.BlockSpec((1, HIDDEN), lambda i: (0, 0)),
            pl.BlockSpec((CLIN_DIM, HIDDEN), lambda i: (0, 0)),
            pl.BlockSpec((HIDDEN, HIDDEN), lambda i: (0, 0)),
            pl.BlockSpec((1, HIDDEN), lambda i: (0, 0)),
            pl.BlockSpec((HIDDEN, HIDDEN), lambda i: (0, 0)),
        ],
        out_specs=[
            pl.BlockSpec((R, HALF), lambda i: (i, 0)),
            pl.BlockSpec((R, HALF), lambda i: (i, 0)),
            pl.BlockSpec((R, 1), lambda i: (i, 0)),
        ],
        out_shape=[
            jax.ShapeDtypeStruct((N_NODES, HALF), jnp.float32),
            jax.ShapeDtypeStruct((N_NODES, HALF), jnp.float32),
            jax.ShapeDtypeStruct((N_NODES, 1), jnp.float32),
        ],
        name="tc_fused_mlp",
    )(mel, clinical, degp3, Wm, bm2, Wcc, Wcm, bc2, W1)


def _tc2_body(sa_ref, sb_ref, hsa_ref, hsb_ref, dinv_ref, b1_ref, w2_ref,
              hs2_ref):
    s1 = jnp.concatenate([sa_ref[...], sb_ref[...]], axis=1)
    hs = jnp.concatenate([hsa_ref[...], hsb_ref[...]], axis=1)
    dinv = dinv_ref[...]
    x2 = jnp.maximum(dinv * (s1 + hs) + b1_ref[...], 0.0)
    h2p = jnp.dot(x2, w2_ref[...], preferred_element_type=jnp.float32)
    hs2_ref[...] = h2p * dinv


def _tc2(s1a, s1b, hs_a, hs_b, dinv, b12, W2p):
    return pl.pallas_call(
        _tc2_body,
        grid=(GRID,),
        in_specs=[
            pl.BlockSpec((R, HALF), lambda i: (i, 0)),
            pl.BlockSpec((R, HALF), lambda i: (i, 0)),
            pl.BlockSpec((R, HALF), lambda i: (i, 0)),
            pl.BlockSpec((R, HALF), lambda i: (i, 0)),
            pl.BlockSpec((R, 1), lambda i: (i, 0)),
            pl.BlockSpec((1, HIDDEN), lambda i: (0, 0)),
            pl.BlockSpec((HIDDEN, L2W), lambda i: (0, 0)),
        ],
        out_specs=pl.BlockSpec((R, L2W), lambda i: (i, 0)),
        out_shape=jax.ShapeDtypeStruct((N_NODES, L2W), jnp.float32),
        name="tc_layer2_prep",
    )(s1a, s1b, hs_a, hs_b, dinv, b12, W2p)


def _tc3_body(s2_ref, hs2_ref, dinv_ref, b2_ref, out_ref):
    tot = s2_ref[0] + s2_ref[1] + hs2_ref[...]
    out_ref[...] = (dinv_ref[...] * tot + b2_ref[...])[:, :NUM_CLASSES]


def _tc3(s2p, hs2p, dinv, b2p):
    return pl.pallas_call(
        _tc3_body,
        grid=(GRID,),
        in_specs=[
            pl.BlockSpec((NC, R, L2W), lambda i: (0, i, 0)),
            pl.BlockSpec((R, L2W), lambda i: (i, 0)),
            pl.BlockSpec((R, 1), lambda i: (i, 0)),
            pl.BlockSpec((1, L2W), lambda i: (0, 0)),
        ],
        out_specs=pl.BlockSpec((R, NUM_CLASSES), lambda i: (i, 0)),
        out_shape=jax.ShapeDtypeStruct((N_NODES, NUM_CLASSES), jnp.float32),
        name="tc_finalize",
    )(s2p, hs2p, dinv, b2p)


def kernel(clinical, mel, edge_index, Wm, bm, Wc, bc, W1, b1, W2, b2):
    src = edge_index[0].astype(jnp.int32)
    dst = edge_index[1].astype(jnp.int32)

    degp = _degree(dst)
    degp3 = degp[:, :N_NODES].reshape(NC, N_NODES, 1)

    bm2 = bm.reshape(1, HIDDEN)
    bc2 = bc.reshape(1, HIDDEN)
    b12 = b1.reshape(1, HIDDEN)
    Wcc = Wc[:CLIN_DIM]
    Wcm = Wc[CLIN_DIM:]
    W2p = jnp.pad(W2, ((0, 0), (0, L2W - NUM_CLASSES)))
    b2p = jnp.pad(b2, (0, L2W - NUM_CLASSES)).reshape(1, L2W)

    hs_a, hs_b, dinv = _tc1(mel, clinical, degp3, Wm, bm2, Wcc, Wcm, bc2, W1)

    s1a, s1b = _message_pass1(src, dst, hs_a, hs_b)
    s1a = s1a[:N_NODES]
    s1b = s1b[:N_NODES]

    hs2p = _tc2(s1a, s1b, hs_a, hs_b, dinv, b12, W2p)

    s2p = _message_pass2(src, dst, hs2p)
    s2p = s2p[:, :N_NODES]

    return _tc3(s2p, hs2p, dinv, b2p)


# speedup vs baseline: 25.9636x; 25.9636x over previous
import functools

import jax
import jax.numpy as jnp
from jax import lax
from jax.experimental import pallas as pl
from jax.experimental.pallas import tpu as pltpu
from jax.experimental.pallas import tpu_sc as plsc

N_NODES = 50000
HIDDEN = 64
NUM_CLASSES = 4
CLIN_DIM = 128
MEL_DIM = 512
N_EDGES = 800000

NC = 2
NS = 16
NPAD = 51200
ROWS_PER_SUB = NPAD // NS
HALF = HIDDEN // 2
L2W = 16

R = 1000
GRID = N_NODES // R


def _mesh():
    return plsc.VectorSubcoreMesh(core_axis_name="c", subcore_axis_name="s")


def _deg_body(dst_hbm, zero1_hbm, out_hbm, acc, idx_v, ones_v, *, epw, chunk):
    c = lax.axis_index("c")
    s = lax.axis_index("s")
    r0 = s * ROWS_PER_SUB
    pltpu.sync_copy(zero1_hbm, acc.at[pl.ds(r0, ROWS_PER_SUB)])
    def fill(i, _):
        ones_v[pl.ds(i * 16, 16)] = jnp.ones((16,), jnp.float32)
        return 0
    lax.fori_loop(0, chunk // 16, fill, 0)
    plsc.subcore_barrier()

    base = (c * NS + s) * epw

    def step(k, _):
        off = base + k * chunk
        pltpu.sync_copy(dst_hbm.at[pl.ds(off, chunk)], idx_v)
        pltpu.sync_copy(ones_v, acc.at[idx_v], add=True)
        return 0
    lax.fori_loop(0, epw // chunk, step, 0)

    plsc.subcore_barrier()
    pltpu.sync_copy(acc.at[pl.ds(r0, ROWS_PER_SUB)],
                    out_hbm.at[c, pl.ds(r0, ROWS_PER_SUB)])


def _degree(dst_i32):
    epw = N_EDGES // (NC * NS)
    chunk = 5000
    body = functools.partial(_deg_body, epw=epw, chunk=chunk)
    f = pl.kernel(
        body,
        out_type=jax.ShapeDtypeStruct((NC, NPAD), jnp.float32),
        mesh=_mesh(),
        scratch_types=[
            pltpu.VMEM_SHARED((NPAD,), jnp.float32),
            pltpu.VMEM((chunk,), jnp.int32),
            pltpu.VMEM((chunk,), jnp.float32),
        ],
        compiler_params=pltpu.CompilerParams(use_tc_tiling_on_sc=False),
        name="sc_degree",
    )
    zero1 = jnp.zeros((ROWS_PER_SUB,), jnp.float32)
    return f(dst_i32, zero1)


def _mp1_body(src_hbm, dst_hbm, hsa_hbm, hsb_hbm, zero2_hbm,
              outa_hbm, outb_hbm,
              acc, src_v, dst_v, rows_v, gsem, *, eps, chunk):
    c = lax.axis_index("c")
    s = lax.axis_index("s")
    r0 = s * ROWS_PER_SUB
    pltpu.sync_copy(zero2_hbm, acc.at[pl.ds(r0, ROWS_PER_SUB)])
    plsc.subcore_barrier()

    base = s * eps
    nsteps = eps // chunk

    def make_step(hs_hbm):
        def step(k, _):
            off = base + k * chunk
            pltpu.sync_copy(src_hbm.at[pl.ds(off, chunk)], src_v)
            pltpu.sync_copy(dst_hbm.at[pl.ds(off, chunk)], dst_v)
            pltpu.async_copy(hs_hbm.at[src_v], rows_v, gsem).wait()
            pltpu.sync_copy(rows_v, acc.at[dst_v], add=True)
            return 0
        return step

    @pl.when(c == 0)
    def _():
        lax.fori_loop(0, nsteps, make_step(hsa_hbm), 0)

    @pl.when(c == 1)
    def _():
        lax.fori_loop(0, nsteps, make_step(hsb_hbm), 0)

    plsc.subcore_barrier()

    @pl.when(c == 0)
    def _():
        pltpu.sync_copy(acc.at[pl.ds(r0, ROWS_PER_SUB)],
                        outa_hbm.at[pl.ds(r0, ROWS_PER_SUB)])

    @pl.when(c == 1)
    def _():
        pltpu.sync_copy(acc.at[pl.ds(r0, ROWS_PER_SUB)],
                        outb_hbm.at[pl.ds(r0, ROWS_PER_SUB)])


def _message_pass1(src_i32, dst_i32, hs_a, hs_b):
    eps = N_EDGES // NS
    chunk = 400
    body = functools.partial(_mp1_body, eps=eps, chunk=chunk)
    f = pl.kernel(
        body,
        out_type=(jax.ShapeDtypeStruct((NPAD, HALF), jnp.float32),
                  jax.ShapeDtypeStruct((NPAD, HALF), jnp.float32)),
        mesh=_mesh(),
        scratch_types=[
            pltpu.VMEM_SHARED((NPAD, HALF), jnp.float32),
            pltpu.VMEM((chunk,), jnp.int32),
            pltpu.VMEM((chunk,), jnp.int32),
            pltpu.VMEM((chunk, HALF), jnp.float32),
            pltpu.SemaphoreType.DMA,
        ],
        compiler_params=pltpu.CompilerParams(use_tc_tiling_on_sc=False),
        name="sc_message_pass1",
    )
    zero2 = jnp.zeros((ROWS_PER_SUB, HALF), jnp.float32)
    return f(src_i32, dst_i32, hs_a, hs_b, zero2)


def _mp2_body(src_hbm, dst_hbm, hs2_hbm, zero3_hbm, out_hbm,
              acc, src_v, dst_v, rows_v, gsem, *, epw, chunk):
    c = lax.axis_index("c")
    s = lax.axis_index("s")
    r0 = s * ROWS_PER_SUB
    pltpu.sync_copy(zero3_hbm, acc.at[pl.ds(r0, ROWS_PER_SUB)])
    plsc.subcore_barrier()

    base = (c * NS + s) * epw

    def step(k, _):
        off = base + k * chunk
        pltpu.sync_copy(src_hbm.at[pl.ds(off, chunk)], src_v)
        pltpu.sync_copy(dst_hbm.at[pl.ds(off, chunk)], dst_v)
        pltpu.async_copy(hs2_hbm.at[src_v], rows_v, gsem).wait()
        pltpu.sync_copy(rows_v, acc.at[dst_v], add=True)
        return 0
    lax.fori_loop(0, epw // chunk, step, 0)

    plsc.subcore_barrier()
    pltpu.sync_copy(acc.at[pl.ds(r0, ROWS_PER_SUB)],
                    out_hbm.at[c, pl.ds(r0, ROWS_PER_SUB)])


def _message_pass2(src_i32, dst_i32, hs2p):
    epw = N_EDGES // (NC * NS)
    chunk = 1000
    body = functools.partial(_mp2_body, epw=epw, chunk=chunk)
    f = pl.kernel(
        body,
        out_type=jax.ShapeDtypeStruct((NC, NPAD, L2W), jnp.float32),
        mesh=_mesh(),
        scratch_types=[
            pltpu.VMEM_SHARED((NPAD, L2W), jnp.float32),
            pltpu.VMEM((chunk,), jnp.int32),
            pltpu.VMEM((chunk,), jnp.int32),
            pltpu.VMEM((chunk, L2W), jnp.float32),
            pltpu.SemaphoreType.DMA,
        ],
        compiler_params=pltpu.CompilerParams(use_tc_tiling_on_sc=False),
        name="sc_message_pass2",
    )
    zero3 = jnp.zeros((ROWS_PER_SUB, L2W), jnp.float32)
    return f(src_i32, dst_i32, hs2p, zero3)


def _tc1_body(mel_ref, clin_ref, degp_ref, wm_ref, bm_ref, wcc_ref, wcm_ref,
              bc_ref, w1_ref, hsa_ref, hsb_ref, dinv_ref):
    m = jnp.maximum(
        jnp.dot(mel_ref[...], wm_ref[...], preferred_element_type=jnp.float32)
        + bm_ref[...], 0.0)
    x = jnp.maximum(
        jnp.dot(clin_ref[...], wcc_ref[...], preferred_element_type=jnp.float32)
        + jnp.dot(m, wcm_ref[...], preferred_element_type=jnp.float32)
        + bc_ref[...], 0.0)
    h1 = jnp.dot(x, w1_ref[...], preferred_element_type=jnp.float32)
    deg = degp_ref[0] + degp_ref[1] + 1.0
    dinv = lax.rsqrt(deg)
    hs = h1 * dinv
    hsa_ref[...] = hs[:, :HALF]
    hsb_ref[...] = hs[:, HALF:]
    dinv_ref[...] = dinv


def _tc1(mel, clinical, degp3, Wm, bm2, Wcc, Wcm, bc2, W1):
    return pl.pallas_call(
        _tc1_body,
        grid=(GRID,),
        in_specs=[
            pl.BlockSpec((R, MEL_DIM), lambda i: (i, 0)),
            pl.BlockSpec((R, CLIN_DIM), lambda i: (i, 0)),
            pl.BlockSpec((NC, R, 1), lambda i: (0, i, 0)),
            pl.BlockSpec((MEL_DIM, HIDDEN), lambda i: (0, 0)),
            pl.BlockSpec((1, HIDDEN), lambda i: (0, 0)),
            pl.BlockSpec((CLIN_DIM, HIDDEN), lambda i: (0, 0)),
            pl.BlockSpec((HIDDEN, HIDDEN), lambda i: (0, 0)),
            pl.BlockSpec((1, HIDDEN), lambda i: (0, 0)),
            pl.BlockSpec((HIDDEN, HIDDEN), lambda i: (0, 0)),
        ],
        out_specs=[
            pl.BlockSpec((R, HALF), lambda i: (i, 0)),
            pl.BlockSpec((R, HALF), lambda i: (i, 0)),
            pl.BlockSpec((R, 1), lambda i: (i, 0)),
        ],
        out_shape=[
            jax.ShapeDtypeStruct((N_NODES, HALF), jnp.float32),
            jax.ShapeDtypeStruct((N_NODES, HALF), jnp.float32),
            jax.ShapeDtypeStruct((N_NODES, 1), jnp.float32),
        ],
        name="tc_fused_mlp",
    )(mel, clinical, degp3, Wm, bm2, Wcc, Wcm, bc2, W1)


def _tc2_body(sa_ref, sb_ref, hsa_ref, hsb_ref, dinv_ref, b1_ref, w2_ref,
              hs2_ref):
    s1 = jnp.concatenate([sa_ref[...], sb_ref[...]], axis=1)
    hs = jnp.concatenate([hsa_ref[...], hsb_ref[...]], axis=1)
    dinv = dinv_ref[...]
    x2 = jnp.maximum(dinv * (s1 + hs) + b1_ref[...], 0.0)
    h2p = jnp.dot(x2, w2_ref[...], preferred_element_type=jnp.float32)
    hs2_ref[...] = h2p * dinv


def _tc2(s1a, s1b, hs_a, hs_b, dinv, b12, W2p):
    return pl.pallas_call(
        _tc2_body,
        grid=(GRID,),
        in_specs=[
            pl.BlockSpec((R, HALF), lambda i: (i, 0)),
            pl.BlockSpec((R, HALF), lambda i: (i, 0)),
            pl.BlockSpec((R, HALF), lambda i: (i, 0)),
            pl.BlockSpec((R, HALF), lambda i: (i, 0)),
            pl.BlockSpec((R, 1), lambda i: (i, 0)),
            pl.BlockSpec((1, HIDDEN), lambda i: (0, 0)),
            pl.BlockSpec((HIDDEN, L2W), lambda i: (0, 0)),
        ],
        out_specs=pl.BlockSpec((R, L2W), lambda i: (i, 0)),
        out_shape=jax.ShapeDtypeStruct((N_NODES, L2W), jnp.float32),
        name="tc_layer2_prep",
    )(s1a, s1b, hs_a, hs_b, dinv, b12, W2p)


def _tc3_body(s2_ref, hs2_ref, dinv_ref, b2_ref, out_ref):
    tot = s2_ref[0] + s2_ref[1] + hs2_ref[...]
    out_ref[...] = (dinv_ref[...] * tot + b2_ref[...])[:, :NUM_CLASSES]


def _tc3(s2p, hs2p, dinv, b2p):
    return pl.pallas_call(
        _tc3_body,
        grid=(GRID,),
        in_specs=[
            pl.BlockSpec((NC, R, L2W), lambda i: (0, i, 0)),
            pl.BlockSpec((R, L2W), lambda i: (i, 0)),
            pl.BlockSpec((R, 1), lambda i: (i, 0)),
            pl.BlockSpec((1, L2W), lambda i: (0, 0)),
        ],
        out_specs=pl.BlockSpec((R, NUM_CLASSES), lambda i: (i, 0)),
        out_shape=jax.ShapeDtypeStruct((N_NODES, NUM_CLASSES), jnp.float32),
        name="tc_finalize",
    )(s2p, hs2p, dinv, b2p)


def kernel(clinical, mel, edge_index, Wm, bm, Wc, bc, W1, b1, W2, b2):
    src = edge_index[0].astype(jnp.int32)
    dst = edge_index[1].astype(jnp.int32)

    degp = _degree(dst)
    degp3 = degp[:, :N_NODES].reshape(NC, N_NODES, 1)

    bm2 = bm.reshape(1, HIDDEN)
    bc2 = bc.reshape(1, HIDDEN)
    b12 = b1.reshape(1, HIDDEN)
    Wcc = Wc[:CLIN_DIM]
    Wcm = Wc[CLIN_DIM:]
    W2p = jnp.pad(W2, ((0, 0), (0, L2W - NUM_CLASSES)))
    b2p = jnp.pad(b2, (0, L2W - NUM_CLASSES)).reshape(1, L2W)

    hs_a, hs_b, dinv = _tc1(mel, clinical, degp3, Wm, bm2, Wcc, Wcm, bc2, W1)

    s1a, s1b = _message_pass1(src, dst, hs_a, hs_b)
    s1a = s1a[:N_NODES]
    s1b = s1b[:N_NODES]

    hs2p = _tc2(s1a, s1b, hs_a, hs_b, dinv, b12, W2p)

    s2p = _message_pass2(src, dst, hs2p)
    s2p = s2p[:, :N_NODES]

    return _tc3(s2p, hs2p, dinv, b2p)


# speedup vs baseline: 32.8710x; 1.2660x over previous
import functools

import jax
import jax.numpy as jnp
from jax import lax
from jax.experimental import pallas as pl
from jax.experimental.pallas import tpu as pltpu
from jax.experimental.pallas import tpu_sc as plsc

N_NODES = 50000
HIDDEN = 64
NUM_CLASSES = 4
CLIN_DIM = 128
MEL_DIM = 512
N_EDGES = 800000

NC = 2
NS = 16
NPAD = 51200
ROWS_PER_SUB = NPAD // NS
QW = 16
NQ = HIDDEN // QW

CHUNK = 1000
NBLK = N_EDGES // CHUNK
NB = 3

R = 2000
GRID = N_NODES // R


def _mesh():
    return plsc.VectorSubcoreMesh(core_axis_name="c", subcore_axis_name="s")


def _ring_loop(hs_hbm, sd_hbm, acc, sdb, rows, gsem, ssem, base_blk, nblk):
    def gather(b):
        return pltpu.make_async_copy(hs_hbm.at[sdb.at[b, 0]], rows.at[b],
                                     gsem.at[b])

    def scatter(b):
        return pltpu.make_async_copy(rows.at[b], acc.at[sdb.at[b, 1]],
                                     ssem.at[b])

    pltpu.sync_copy(sd_hbm.at[base_blk], sdb.at[0])
    gather(0).start()
    pltpu.sync_copy(sd_hbm.at[base_blk + 1], sdb.at[1])
    gather(1).start()

    def step(j, _):
        b = lax.rem(j, NB)
        b2 = lax.rem(j + 2, NB)
        gather(b).wait()
        pltpu.async_copy(rows.at[b], acc.at[sdb.at[b, 1]], ssem.at[b],
                         add=True)

        @pl.when(jnp.logical_and(j >= 1, j < nblk - 2))
        def _():
            scatter(b2).wait()

        @pl.when(j < nblk - 2)
        def _():
            pltpu.sync_copy(sd_hbm.at[base_blk + j + 2], sdb.at[b2])
            gather(b2).start()
        return 0

    lax.fori_loop(0, nblk, step, 0)
    scatter((nblk - 3) % NB).wait()
    scatter((nblk - 2) % NB).wait()
    scatter((nblk - 1) % NB).wait()


def _deg_body(sd_hbm, zero1_hbm, ones_hbm, out_hbm, acc, idx_v, ones_v):
    c = lax.axis_index("c")
    s = lax.axis_index("s")
    r0 = s * ROWS_PER_SUB
    pltpu.sync_copy(zero1_hbm, acc.at[pl.ds(r0, ROWS_PER_SUB)])
    pltpu.sync_copy(ones_hbm, ones_v)
    plsc.subcore_barrier()

    nblk_w = NBLK // (NC * NS)
    sb = 5
    base = (c * NS + s) * nblk_w

    def step(k, _):
        pltpu.sync_copy(sd_hbm.at[pl.ds(base + k * sb, sb)], idx_v)
        for i in range(sb):
            pltpu.sync_copy(ones_v, acc.at[idx_v.at[i, 1]], add=True)
        return 0
    lax.fori_loop(0, nblk_w // sb, step, 0)

    plsc.subcore_barrier()
    pltpu.sync_copy(acc.at[pl.ds(r0, ROWS_PER_SUB)],
                    out_hbm.at[c, pl.ds(r0, ROWS_PER_SUB)])


def _degree(sd):
    f = pl.kernel(
        _deg_body,
        out_type=jax.ShapeDtypeStruct((NC, NPAD), jnp.float32),
        mesh=_mesh(),
        scratch_types=[
            pltpu.VMEM_SHARED((NPAD,), jnp.float32),
            pltpu.VMEM((5, 2, CHUNK), jnp.int32),
            pltpu.VMEM((CHUNK,), jnp.float32),
        ],
        compiler_params=pltpu.CompilerParams(use_tc_tiling_on_sc=False),
        name="sc_degree",
    )
    zero1 = jnp.zeros((ROWS_PER_SUB,), jnp.float32)
    ones = jnp.ones((CHUNK,), jnp.float32)
    return f(sd, zero1, ones)


def _mp1_body(sd_hbm, hq0_hbm, hq1_hbm, hq2_hbm, hq3_hbm, zero2_hbm,
              o0_hbm, o1_hbm, o2_hbm, o3_hbm,
              acc, sdb, rows, gsem, ssem):
    c = lax.axis_index("c")
    s = lax.axis_index("s")
    r0 = s * ROWS_PER_SUB
    rows_slice = pl.ds(r0, ROWS_PER_SUB)
    pltpu.sync_copy(zero2_hbm, acc.at[rows_slice])
    plsc.subcore_barrier()

    nblk_s = NBLK // NS
    base = s * nblk_s

    def quarters(hqa, hqb, oa, ob):
        _ring_loop(hqa, sd_hbm, acc, sdb, rows, gsem, ssem, base, nblk_s)
        plsc.subcore_barrier()
        pltpu.sync_copy(acc.at[rows_slice], oa.at[rows_slice])
        pltpu.sync_copy(zero2_hbm, acc.at[rows_slice])
        plsc.subcore_barrier()
        _ring_loop(hqb, sd_hbm, acc, sdb, rows, gsem, ssem, base, nblk_s)
        plsc.subcore_barrier()
        pltpu.sync_copy(acc.at[rows_slice], ob.at[rows_slice])

    @pl.when(c == 0)
    def _():
        quarters(hq0_hbm, hq1_hbm, o0_hbm, o1_hbm)

    @pl.when(c == 1)
    def _():
        quarters(hq2_hbm, hq3_hbm, o2_hbm, o3_hbm)


def _message_pass1(sd, hq0, hq1, hq2, hq3):
    f = pl.kernel(
        _mp1_body,
        out_type=tuple(jax.ShapeDtypeStruct((NPAD, QW), jnp.float32)
                       for _ in range(NQ)),
        mesh=_mesh(),
        scratch_types=[
            pltpu.VMEM_SHARED((NPAD, QW), jnp.float32),
            pltpu.VMEM((NB, 2, CHUNK), jnp.int32),
            pltpu.VMEM((NB, CHUNK, QW), jnp.float32),
            pltpu.SemaphoreType.DMA((NB,)),
            pltpu.SemaphoreType.DMA((NB,)),
        ],
        compiler_params=pltpu.CompilerParams(use_tc_tiling_on_sc=False),
        name="sc_message_pass1",
    )
    zero2 = jnp.zeros((ROWS_PER_SUB, QW), jnp.float32)
    return f(sd, hq0, hq1, hq2, hq3, zero2)


def _mp2_body(sd_hbm, hs2_hbm, zero2_hbm, out_hbm,
              acc, sdb, rows, gsem, ssem):
    c = lax.axis_index("c")
    s = lax.axis_index("s")
    r0 = s * ROWS_PER_SUB
    rows_slice = pl.ds(r0, ROWS_PER_SUB)
    pltpu.sync_copy(zero2_hbm, acc.at[rows_slice])
    plsc.subcore_barrier()

    nblk_w = NBLK // (NC * NS)
    base = (c * NS + s) * nblk_w
    _ring_loop(hs2_hbm, sd_hbm, acc, sdb, rows, gsem, ssem, base, nblk_w)

    plsc.subcore_barrier()
    pltpu.sync_copy(acc.at[rows_slice], out_hbm.at[c, rows_slice])


def _message_pass2(sd, hs2p):
    f = pl.kernel(
        _mp2_body,
        out_type=jax.ShapeDtypeStruct((NC, NPAD, QW), jnp.float32),
        mesh=_mesh(),
        scratch_types=[
            pltpu.VMEM_SHARED((NPAD, QW), jnp.float32),
            pltpu.VMEM((NB, 2, CHUNK), jnp.int32),
            pltpu.VMEM((NB, CHUNK, QW), jnp.float32),
            pltpu.SemaphoreType.DMA((NB,)),
            pltpu.SemaphoreType.DMA((NB,)),
        ],
        compiler_params=pltpu.CompilerParams(use_tc_tiling_on_sc=False),
        name="sc_message_pass2",
    )
    zero2 = jnp.zeros((ROWS_PER_SUB, QW), jnp.float32)
    return f(sd, hs2p, zero2)


def _tc1_body(mel_ref, clin_ref, degp_ref, wm_ref, bm_ref, wcc_ref, wcm_ref,
              bc_ref, w1_ref, hq0_ref, hq1_ref, hq2_ref, hq3_ref, dinv_ref):
    m = jnp.maximum(
        jnp.dot(mel_ref[...], wm_ref[...], preferred_element_type=jnp.float32)
        + bm_ref[...], 0.0)
    x = jnp.maximum(
        jnp.dot(clin_ref[...], wcc_ref[...], preferred_element_type=jnp.float32)
        + jnp.dot(m, wcm_ref[...], preferred_element_type=jnp.float32)
        + bc_ref[...], 0.0)
    h1 = jnp.dot(x, w1_ref[...], preferred_element_type=jnp.float32)
    deg = degp_ref[0] + degp_ref[1] + 1.0
    dinv = lax.rsqrt(deg)
    hs = h1 * dinv
    hq0_ref[...] = hs[:, 0 * QW:1 * QW]
    hq1_ref[...] = hs[:, 1 * QW:2 * QW]
    hq2_ref[...] = hs[:, 2 * QW:3 * QW]
    hq3_ref[...] = hs[:, 3 * QW:4 * QW]
    dinv_ref[...] = dinv


def _tc1(mel, clinical, degp3, Wm, bm2, Wcc, Wcm, bc2, W1):
    return pl.pallas_call(
        _tc1_body,
        grid=(GRID,),
        in_specs=[
            pl.BlockSpec((R, MEL_DIM), lambda i: (i, 0)),
            pl.BlockSpec((R, CLIN_DIM), lambda i: (i, 0)),
            pl.BlockSpec((NC, R, 1), lambda i: (0, i, 0)),
            pl.BlockSpec((MEL_DIM, HIDDEN), lambda i: (0, 0)),
            pl.BlockSpec((1, HIDDEN), lambda i: (0, 0)),
            pl.BlockSpec((CLIN_DIM, HIDDEN), lambda i: (0, 0)),
            pl.BlockSpec((HIDDEN, HIDDEN), lambda i: (0, 0)),
            pl.BlockSpec((1, HIDDEN), lambda i: (0, 0)),
            pl.BlockSpec((HIDDEN, HIDDEN), lambda i: (0, 0)),
        ],
        out_specs=[pl.BlockSpec((R, QW), lambda i: (i, 0))] * NQ
        + [pl.BlockSpec((R, 1), lambda i: (i, 0))],
        out_shape=[jax.ShapeDtypeStruct((N_NODES, QW), jnp.float32)] * NQ
        + [jax.ShapeDtypeStruct((N_NODES, 1), jnp.float32)],
        name="tc_fused_mlp",
    )(mel, clinical, degp3, Wm, bm2, Wcc, Wcm, bc2, W1)


def _tc2_body(s0_ref, s1_ref, s2_ref, s3_ref,
              h0_ref, h1_ref, h2_ref, h3_ref,
              dinv_ref, b1_ref, w2_ref, hs2_ref):
    s1 = jnp.concatenate(
        [s0_ref[...], s1_ref[...], s2_ref[...], s3_ref[...]], axis=1)
    hs = jnp.concatenate(
        [h0_ref[...], h1_ref[...], h2_ref[...], h3_ref[...]], axis=1)
    dinv = dinv_ref[...]
    x2 = jnp.maximum(dinv * (s1 + hs) + b1_ref[...], 0.0)
    h2p = jnp.dot(x2, w2_ref[...], preferred_element_type=jnp.float32)
    hs2_ref[...] = h2p * dinv


def _tc2(sq, hq, dinv, b12, W2p):
    return pl.pallas_call(
        _tc2_body,
        grid=(GRID,),
        in_specs=[pl.BlockSpec((R, QW), lambda i: (i, 0))] * (2 * NQ)
        + [
            pl.BlockSpec((R, 1), lambda i: (i, 0)),
            pl.BlockSpec((1, HIDDEN), lambda i: (0, 0)),
            pl.BlockSpec((HIDDEN, QW), lambda i: (0, 0)),
        ],
        out_specs=pl.BlockSpec((R, QW), lambda i: (i, 0)),
        out_shape=jax.ShapeDtypeStruct((N_NODES, QW), jnp.float32),
        name="tc_layer2_prep",
    )(*sq, *hq, dinv, b12, W2p)


def _tc3_body(s2_ref, hs2_ref, dinv_ref, b2_ref, out_ref):
    tot = s2_ref[0] + s2_ref[1] + hs2_ref[...]
    out_ref[...] = (dinv_ref[...] * tot + b2_ref[...])[:, :NUM_CLASSES]


def _tc3(s2p, hs2p, dinv, b2p):
    return pl.pallas_call(
        _tc3_body,
        grid=(GRID,),
        in_specs=[
            pl.BlockSpec((NC, R, QW), lambda i: (0, i, 0)),
            pl.BlockSpec((R, QW), lambda i: (i, 0)),
            pl.BlockSpec((R, 1), lambda i: (i, 0)),
            pl.BlockSpec((1, QW), lambda i: (0, 0)),
        ],
        out_specs=pl.BlockSpec((R, NUM_CLASSES), lambda i: (i, 0)),
        out_shape=jax.ShapeDtypeStruct((N_NODES, NUM_CLASSES), jnp.float32),
        name="tc_finalize",
    )(s2p, hs2p, dinv, b2p)


def kernel(clinical, mel, edge_index, Wm, bm, Wc, bc, W1, b1, W2, b2):
    ei = edge_index.astype(jnp.int32)
    sd = jnp.stack(
        [ei[0].reshape(NBLK, CHUNK), ei[1].reshape(NBLK, CHUNK)], axis=1)

    degp = _degree(sd)
    degp3 = degp[:, :N_NODES].reshape(NC, N_NODES, 1)

    bm2 = bm.reshape(1, HIDDEN)
    bc2 = bc.reshape(1, HIDDEN)
    b12 = b1.reshape(1, HIDDEN)
    Wcc = Wc[:CLIN_DIM]
    Wcm = Wc[CLIN_DIM:]
    W2p = jnp.pad(W2, ((0, 0), (0, QW - NUM_CLASSES)))
    b2p = jnp.pad(b2, (0, QW - NUM_CLASSES)).reshape(1, QW)

    *hq, dinv = _tc1(mel, clinical, degp3, Wm, bm2, Wcc, Wcm, bc2, W1)

    sq = _message_pass1(sd, *hq)

    hs2p = _tc2(sq, hq, dinv, b12, W2p)

    s2p = _message_pass2(sd, hs2p)

    return _tc3(s2p, hs2p, dinv, b2p)


# speedup vs baseline: 43.1529x; 1.3128x over previous
import jax
import jax.numpy as jnp
from jax import lax
from jax.experimental import pallas as pl
from jax.experimental.pallas import tpu as pltpu
from jax.experimental.pallas import tpu_sc as plsc

N_NODES = 50000
HIDDEN = 64
NUM_CLASSES = 4
CLIN_DIM = 128
MEL_DIM = 512
N_EDGES = 800000

NC = 2
NS = 16
NPAD = 51200
ROWS_PER_SUB = NPAD // NS
QW = 16
NQ = HIDDEN // QW

CHUNK = 1000
CPAD = 1008
NBLK = N_EDGES // CHUNK
NB = 4

R = 2000
GRID = N_NODES // R


def _mesh():
    return plsc.VectorSubcoreMesh(core_axis_name="c", subcore_axis_name="s")


def _ring_loop(table_hbm, splane, dplane, acc, sbuf, dbuf, rows,
               srcsem, dstsem, gsem, csem, base_blk, nblk, q):
    csl = pl.ds(0, CHUNK)

    def src_load(b, g):
        pltpu.async_copy(splane.at[g], sbuf.at[b, csl], srcsem.at[b])

    def dst_load(b, g):
        pltpu.async_copy(dplane.at[g], dbuf.at[b], dstsem.at[b])

    def src_wait(b):
        pltpu.make_async_copy(splane.at[base_blk], sbuf.at[b, csl],
                              srcsem.at[b]).wait()

    def dst_wait(b):
        pltpu.make_async_copy(dplane.at[base_blk], dbuf.at[b],
                              dstsem.at[b]).wait()

    def gather(b):
        return pltpu.make_async_copy(table_hbm.at[sbuf.at[b, csl]],
                                     rows.at[b], gsem.at[b])

    def scat(b):
        return pltpu.make_async_copy(rows.at[b], acc.at[dbuf.at[b]],
                                     csem.at[b])

    def transform(b):
        if q is not None:
            for i in range(CPAD // 16):
                sl = pl.ds(i * 16, 16)
                sbuf[b, sl] = sbuf[b, sl] * 4 + q

    for t in range(3):
        src_load(t, base_blk + t)
        dst_load(t, base_blk + t)
    for t in range(2):
        src_wait(t)
        transform(t)
        gather(t).start()

    def step(j, _):
        b = lax.rem(j, NB)
        bg = lax.rem(j + 2, NB)
        bs = lax.rem(j + 3, NB)
        dst_wait(b)
        gather(b).wait()
        pltpu.async_copy(rows.at[b], acc.at[dbuf.at[b]], csem.at[b],
                         add=True)

        @pl.when(jnp.logical_and(j >= 1, j + 3 < nblk))
        def _():
            scat(bs).wait()

        @pl.when(j + 3 < nblk)
        def _():
            src_load(bs, base_blk + j + 3)
            dst_load(bs, base_blk + j + 3)

        @pl.when(j + 2 < nblk)
        def _():
            src_wait(bg)
            transform(bg)
            gather(bg).start()
        return 0

    lax.fori_loop(0, nblk, step, 0)
    for k in range(4):
        scat((nblk - 4 + k) % NB).wait()


def _deg_body(sd_hbm, zero1_hbm, ones_hbm, out_hbm, acc, idx_v, ones_v):
    c = lax.axis_index("c")
    s = lax.axis_index("s")
    r0 = s * ROWS_PER_SUB
    pltpu.sync_copy(zero1_hbm, acc.at[pl.ds(r0, ROWS_PER_SUB)])
    pltpu.sync_copy(ones_hbm, ones_v)
    plsc.subcore_barrier()

    nblk_w = NBLK // (NC * NS)
    sb = 5
    base = (c * NS + s) * nblk_w

    def step(k, _):
        pltpu.sync_copy(sd_hbm.at[1, pl.ds(base + k * sb, sb)], idx_v)
        for i in range(sb):
            pltpu.sync_copy(ones_v, acc.at[idx_v.at[i]], add=True)
        return 0
    lax.fori_loop(0, nblk_w // sb, step, 0)

    plsc.subcore_barrier()
    pltpu.sync_copy(acc.at[pl.ds(r0, ROWS_PER_SUB)],
                    out_hbm.at[c, pl.ds(r0, ROWS_PER_SUB)])


def _degree(sd):
    f = pl.kernel(
        _deg_body,
        out_type=jax.ShapeDtypeStruct((NC, NPAD), jnp.float32),
        mesh=_mesh(),
        scratch_types=[
            pltpu.VMEM_SHARED((NPAD,), jnp.float32),
            pltpu.VMEM((5, CHUNK), jnp.int32),
            pltpu.VMEM((CHUNK,), jnp.float32),
        ],
        compiler_params=pltpu.CompilerParams(use_tc_tiling_on_sc=False),
        name="sc_degree",
    )
    zero1 = jnp.zeros((ROWS_PER_SUB,), jnp.float32)
    ones = jnp.ones((CHUNK,), jnp.float32)
    return f(sd, zero1, ones)


def _mp1_body(sd_hbm, hs4_hbm, zero2_hbm, out_hbm,
              acc, sbuf, dbuf, rows, srcsem, dstsem, gsem, csem):
    c = lax.axis_index("c")
    s = lax.axis_index("s")
    r0 = s * ROWS_PER_SUB
    rows_slice = pl.ds(r0, ROWS_PER_SUB)
    pltpu.sync_copy(zero2_hbm, acc.at[rows_slice])
    plsc.subcore_barrier()

    nblk_s = NBLK // NS
    base = s * nblk_s
    splane = sd_hbm.at[0]
    dplane = sd_hbm.at[1]

    def run_quarter(q, out_plane):
        _ring_loop(hs4_hbm, splane, dplane, acc, sbuf, dbuf, rows,
                   srcsem, dstsem, gsem, csem, base, nblk_s, q)
        plsc.subcore_barrier()
        pltpu.sync_copy(acc.at[rows_slice], out_hbm.at[out_plane, rows_slice])

    def run_core(qa, qb):
        run_quarter(qa, qa)
        pltpu.sync_copy(zero2_hbm, acc.at[rows_slice])
        plsc.subcore_barrier()
        run_quarter(qb, qb)

    @pl.when(c == 0)
    def _():
        run_core(0, 1)

    @pl.when(c == 1)
    def _():
        run_core(2, 3)


def _message_pass1(sd, hs4):
    f = pl.kernel(
        _mp1_body,
        out_type=jax.ShapeDtypeStruct((NQ, NPAD, QW), jnp.float32),
        mesh=_mesh(),
        scratch_types=[
            pltpu.VMEM_SHARED((NPAD, QW), jnp.float32),
            pltpu.VMEM((NB, CPAD), jnp.int32),
            pltpu.VMEM((NB, CHUNK), jnp.int32),
            pltpu.VMEM((NB, CHUNK, QW), jnp.float32),
            pltpu.SemaphoreType.DMA((NB,)),
            pltpu.SemaphoreType.DMA((NB,)),
            pltpu.SemaphoreType.DMA((NB,)),
            pltpu.SemaphoreType.DMA((NB,)),
        ],
        compiler_params=pltpu.CompilerParams(use_tc_tiling_on_sc=False),
        name="sc_message_pass1",
    )
    zero2 = jnp.zeros((ROWS_PER_SUB, QW), jnp.float32)
    return f(sd, hs4, zero2)


def _mp2_body(sd_hbm, hs2_hbm, zero2_hbm, out_hbm,
              acc, sbuf, dbuf, rows, srcsem, dstsem, gsem, csem):
    c = lax.axis_index("c")
    s = lax.axis_index("s")
    r0 = s * ROWS_PER_SUB
    rows_slice = pl.ds(r0, ROWS_PER_SUB)
    pltpu.sync_copy(zero2_hbm, acc.at[rows_slice])
    plsc.subcore_barrier()

    nblk_w = NBLK // (NC * NS)
    base = (c * NS + s) * nblk_w
    _ring_loop(hs2_hbm, sd_hbm.at[0], sd_hbm.at[1], acc, sbuf, dbuf, rows,
               srcsem, dstsem, gsem, csem, base, nblk_w, None)

    plsc.subcore_barrier()
    pltpu.sync_copy(acc.at[rows_slice], out_hbm.at[c, rows_slice])


def _message_pass2(sd, hs2p):
    f = pl.kernel(
        _mp2_body,
        out_type=jax.ShapeDtypeStruct((NC, NPAD, QW), jnp.float32),
        mesh=_mesh(),
        scratch_types=[
            pltpu.VMEM_SHARED((NPAD, QW), jnp.float32),
            pltpu.VMEM((NB, CPAD), jnp.int32),
            pltpu.VMEM((NB, CHUNK), jnp.int32),
            pltpu.VMEM((NB, CHUNK, QW), jnp.float32),
            pltpu.SemaphoreType.DMA((NB,)),
            pltpu.SemaphoreType.DMA((NB,)),
            pltpu.SemaphoreType.DMA((NB,)),
            pltpu.SemaphoreType.DMA((NB,)),
        ],
        compiler_params=pltpu.CompilerParams(use_tc_tiling_on_sc=False),
        name="sc_message_pass2",
    )
    zero2 = jnp.zeros((ROWS_PER_SUB, QW), jnp.float32)
    return f(sd, hs2p, zero2)


def _tc1_body(mel_ref, clin_ref, degp_ref, wm_ref, bm_ref, wcc_ref, wcm_ref,
              bc_ref, w1_ref, hs_ref, dinv_ref):
    m = jnp.maximum(
        jnp.dot(mel_ref[...], wm_ref[...], preferred_element_type=jnp.float32)
        + bm_ref[...], 0.0)
    x = jnp.maximum(
        jnp.dot(clin_ref[...], wcc_ref[...], preferred_element_type=jnp.float32)
        + jnp.dot(m, wcm_ref[...], preferred_element_type=jnp.float32)
        + bc_ref[...], 0.0)
    h1 = jnp.dot(x, w1_ref[...], preferred_element_type=jnp.float32)
    deg = degp_ref[0] + degp_ref[1] + 1.0
    dinv = lax.rsqrt(deg)
    hs_ref[...] = h1 * dinv
    dinv_ref[...] = dinv


def _tc1(mel, clinical, degp3, Wm, bm2, Wcc, Wcm, bc2, W1):
    return pl.pallas_call(
        _tc1_body,
        grid=(GRID,),
        in_specs=[
            pl.BlockSpec((R, MEL_DIM), lambda i: (i, 0)),
            pl.BlockSpec((R, CLIN_DIM), lambda i: (i, 0)),
            pl.BlockSpec((NC, R, 1), lambda i: (0, i, 0)),
            pl.BlockSpec((MEL_DIM, HIDDEN), lambda i: (0, 0)),
            pl.BlockSpec((1, HIDDEN), lambda i: (0, 0)),
            pl.BlockSpec((CLIN_DIM, HIDDEN), lambda i: (0, 0)),
            pl.BlockSpec((HIDDEN, HIDDEN), lambda i: (0, 0)),
            pl.BlockSpec((1, HIDDEN), lambda i: (0, 0)),
            pl.BlockSpec((HIDDEN, HIDDEN), lambda i: (0, 0)),
        ],
        out_specs=[
            pl.BlockSpec((R, HIDDEN), lambda i: (i, 0)),
            pl.BlockSpec((R, 1), lambda i: (i, 0)),
        ],
        out_shape=[
            jax.ShapeDtypeStruct((N_NODES, HIDDEN), jnp.float32),
            jax.ShapeDtypeStruct((N_NODES, 1), jnp.float32),
        ],
        name="tc_fused_mlp",
    )(mel, clinical, degp3, Wm, bm2, Wcc, Wcm, bc2, W1)


def _tc2_body(sq_ref, hs_ref, dinv_ref, b1_ref, w2_ref, hs2_ref):
    s1 = jnp.concatenate(
        [sq_ref[0], sq_ref[1], sq_ref[2], sq_ref[3]], axis=1)
    dinv = dinv_ref[...]
    x2 = jnp.maximum(dinv * (s1 + hs_ref[...]) + b1_ref[...], 0.0)
    h2p = jnp.dot(x2, w2_ref[...], preferred_element_type=jnp.float32)
    hs2_ref[...] = h2p * dinv


def _tc2(sq, hs, dinv, b12, W2p):
    return pl.pallas_call(
        _tc2_body,
        grid=(GRID,),
        in_specs=[
            pl.BlockSpec((NQ, R, QW), lambda i: (0, i, 0)),
            pl.BlockSpec((R, HIDDEN), lambda i: (i, 0)),
            pl.BlockSpec((R, 1), lambda i: (i, 0)),
            pl.BlockSpec((1, HIDDEN), lambda i: (0, 0)),
            pl.BlockSpec((HIDDEN, QW), lambda i: (0, 0)),
        ],
        out_specs=pl.BlockSpec((R, QW), lambda i: (i, 0)),
        out_shape=jax.ShapeDtypeStruct((N_NODES, QW), jnp.float32),
        name="tc_layer2_prep",
    )(sq, hs, dinv, b12, W2p)


def _tc3_body(s2_ref, hs2_ref, dinv_ref, b2_ref, out_ref):
    tot = s2_ref[0] + s2_ref[1] + hs2_ref[...]
    out_ref[...] = (dinv_ref[...] * tot + b2_ref[...])[:, :NUM_CLASSES]


def _tc3(s2p, hs2p, dinv, b2p):
    return pl.pallas_call(
        _tc3_body,
        grid=(GRID,),
        in_specs=[
            pl.BlockSpec((NC, R, QW), lambda i: (0, i, 0)),
            pl.BlockSpec((R, QW), lambda i: (i, 0)),
            pl.BlockSpec((R, 1), lambda i: (i, 0)),
            pl.BlockSpec((1, QW), lambda i: (0, 0)),
        ],
        out_specs=pl.BlockSpec((R, NUM_CLASSES), lambda i: (i, 0)),
        out_shape=jax.ShapeDtypeStruct((N_NODES, NUM_CLASSES), jnp.float32),
        name="tc_finalize",
    )(s2p, hs2p, dinv, b2p)


def kernel(clinical, mel, edge_index, Wm, bm, Wc, bc, W1, b1, W2, b2):
    sd = edge_index.astype(jnp.int32).reshape(2, NBLK, CHUNK)

    degp = _degree(sd)
    degp3 = degp[:, :N_NODES].reshape(NC, N_NODES, 1)

    bm2 = bm.reshape(1, HIDDEN)
    bc2 = bc.reshape(1, HIDDEN)
    b12 = b1.reshape(1, HIDDEN)
    Wcc = Wc[:CLIN_DIM]
    Wcm = Wc[CLIN_DIM:]
    W2p = jnp.pad(W2, ((0, 0), (0, QW - NUM_CLASSES)))
    b2p = jnp.pad(b2, (0, QW - NUM_CLASSES)).reshape(1, QW)

    hs, dinv = _tc1(mel, clinical, degp3, Wm, bm2, Wcc, Wcm, bc2, W1)
    hs4 = hs.reshape(NQ * N_NODES, QW)

    sq = _message_pass1(sd, hs4)

    hs2p = _tc2(sq, hs, dinv, b12, W2p)

    s2p = _message_pass2(sd, hs2p)

    return _tc3(s2p, hs2p, dinv, b2p)
